# Initial kernel scaffold; baseline (speedup 1.0000x reference)
import jax
import jax.numpy as jnp
from jax.experimental import pallas as pl


def kernel(p, x, o):
    raise NotImplementedError("write your pallas kernel here")


# speedup vs baseline: 5.9973x; 5.9973x over previous
import functools

import jax
import jax.numpy as jnp
from jax import lax
from jax.experimental import pallas as pl
from jax.experimental.pallas import tpu as pltpu
from jax.experimental.pallas import tpu_sc as plsc

N = 8192
C = 256
K = 16
BR = 256
GRID = N // BR


def _topk_body(pp_ref, ptT_ref, idx_ref, d_ref):
    xi = pp_ref[:, 0:1]
    yi = pp_ref[:, 1:2]
    zi = pp_ref[:, 2:3]
    xj = ptT_ref[0:1, :]
    yj = ptT_ref[1:2, :]
    zj = ptT_ref[2:3, :]
    sqi = (xi * xi + zi * zi) + yi * yi
    sqj = (xj * xj + zj * zj) + yj * yj
    dot = jnp.dot(pp_ref[:, 0:3], ptT_ref[0:3, :],
                  preferred_element_type=jnp.float32)
    d_ref[...] = (sqi + sqj) - 2.0 * dot
    colio = lax.broadcasted_iota(jnp.int32, (BR, N), 1)
    big = jnp.int32(N)
    cols = []
    for _ in range(K):
        d = d_ref[...]
        m = jnp.min(d, axis=1, keepdims=True)
        j = jnp.min(jnp.where(d == m, colio, big), axis=1, keepdims=True)
        cols.append(j)
        d_ref[...] = jnp.where(colio == j, jnp.float32(jnp.inf), d)
    idx_ref[...] = jnp.concatenate(cols, axis=1)


def _topk(pp, ptT):
    return pl.pallas_call(
        _topk_body,
        grid=(GRID,),
        in_specs=[
            pl.BlockSpec((BR, 128), lambda i: (i, 0)),
            pl.BlockSpec((8, N), lambda i: (0, 0)),
        ],
        out_specs=pl.BlockSpec((BR, K), lambda i: (i, 0)),
        out_shape=jax.ShapeDtypeStruct((N, K), jnp.int32),
        scratch_shapes=[pltpu.VMEM((BR, N), jnp.float32)],
    )(pp, ptT)


_NC, _NS = 2, 16
NW = _NC * _NS
B = N * K
BPW = B // NW
CH = 128
NCH = BPW // CH

def _sc_gather_body(x_hbm, pp16_hbm, idx_hbm, xk_out, pr_out,
                    idx_v, xrows, prows, cent, sem1, sem2):
    wid = lax.axis_index("s") * _NC + lax.axis_index("c")
    base = wid * BPW
    pltpu.sync_copy(idx_hbm.at[pl.ds(base, BPW)], idx_v)

    def chunk(ci, carry):
        off = ci * CH
        gidx = idx_v.at[pl.ds(off, CH)]
        cp1 = pltpu.async_copy(x_hbm.at[gidx], xrows, sem1)
        cp2 = pltpu.async_copy(pp16_hbm.at[gidx], prows, sem2)
        pt0 = pl.multiple_of((base + off) // K, CH // K)
        pltpu.sync_copy(pp16_hbm.at[pl.ds(pt0, CH // K)], cent)
        cp1.wait()
        cp2.wait()

        def sub1(i, c2):
            cv = cent[lax.div(i, K)]
            prows[i] = prows[i] - cv
            return c2

        lax.fori_loop(0, CH, sub1, 0)
        pltpu.sync_copy(xrows, xk_out.at[pl.ds(base + off, CH)])
        pltpu.sync_copy(prows, pr_out.at[pl.ds(base + off, CH)])
        return carry

    lax.fori_loop(0, NCH, chunk, 0)


@functools.lru_cache(maxsize=1)
def _sc_gather_fn():
    mesh = plsc.VectorSubcoreMesh(core_axis_name="c", subcore_axis_name="s")
    return pl.kernel(
        _sc_gather_body,
        mesh=mesh,
        compiler_params=pltpu.CompilerParams(use_tc_tiling_on_sc=False),
        out_type=[
            jax.ShapeDtypeStruct((B, C), jnp.float32),
            jax.ShapeDtypeStruct((B, 16), jnp.float32),
        ],
        scratch_types=[
            pltpu.VMEM((BPW,), jnp.int32),
            pltpu.VMEM((CH, C), jnp.float32),
            pltpu.VMEM((CH, 16), jnp.float32),
            pltpu.VMEM((CH // K, 16), jnp.float32),
            pltpu.SemaphoreType.DMA,
            pltpu.SemaphoreType.DMA,
        ],
    )


def kernel(p, x, o):
    del o
    pp = jnp.zeros((N, 128), jnp.float32).at[:, :3].set(p)
    ptT = jnp.zeros((8, N), jnp.float32).at[:3, :].set(p.T)
    knn_idx = _topk(pp, ptT)
    pp16 = jnp.zeros((N, 16), jnp.float32).at[:, :3].set(p)
    idx_flat = knn_idx.reshape(B)
    xk, prp = _sc_gather_fn()(x, pp16, idx_flat)
    x_knn = xk.reshape(N, K, C)
    p_r = prp[:, :3].reshape(N, K, 3)
    return (x, x_knn, knn_idx, p_r)
